# baseline (speedup 1.0000x reference)
import jax
import jax.numpy as jnp
from jax.experimental import pallas as pl
from jax.experimental.pallas import tpu as pltpu

_EMBED_DIM = 128
_RULES_SIZE = 1000
_ROWS_PER_BLOCK = 8


_CHUNKS = (256, 256, 256, 232)


def _fused_kernel(ident_ref, row_ref, wt_hbm, b_ref, out_ref, wt_vmem, *sems):
    copies = []
    off = 0
    for n, sem in zip(_CHUNKS, sems):
        c = pltpu.make_async_copy(
            wt_hbm.at[pl.ds(off, n), :], wt_vmem.at[pl.ds(off, n), :], sem)
        c.start()
        copies.append((off, n, c))
        off += n
    r = ident_ref[0] % _ROWS_PER_BLOCK
    row = row_ref[pl.ds(r, 1), :]
    for off, n, c in copies:
        c.wait()
        acc = jax.lax.dot_general(
            row, wt_vmem[pl.ds(off, n), :],
            dimension_numbers=(((1,), (1,)), ((), ())),
            preferred_element_type=jnp.float32,
        )
        out_ref[:, pl.ds(off, n)] = jnp.maximum(
            acc + b_ref[pl.ds(off, n)], 0.0)


def kernel(ident, table, W, b):
    ident = ident.astype(jnp.int32)
    wt = pltpu.with_memory_space_constraint(W.T, pltpu.MemorySpace.HBM)
    grid_spec = pltpu.PrefetchScalarGridSpec(
        num_scalar_prefetch=1,
        grid=(1,),
        in_specs=[
            pl.BlockSpec(
                (_ROWS_PER_BLOCK, _EMBED_DIM),
                lambda i, ident_ref: (ident_ref[0] // _ROWS_PER_BLOCK, 0),
            ),
            pl.BlockSpec(memory_space=pltpu.MemorySpace.HBM),
            pl.BlockSpec((_RULES_SIZE,), lambda i, ident_ref: (0,)),
        ],
        out_specs=pl.BlockSpec((1, _RULES_SIZE), lambda i, ident_ref: (0, 0)),
        scratch_shapes=[
            pltpu.VMEM((_RULES_SIZE, _EMBED_DIM), jnp.float32),
            pltpu.SemaphoreType.DMA,
            pltpu.SemaphoreType.DMA,
            pltpu.SemaphoreType.DMA,
            pltpu.SemaphoreType.DMA,
        ],
    )
    return pl.pallas_call(
        _fused_kernel,
        grid_spec=grid_spec,
        out_shape=jax.ShapeDtypeStruct((1, _RULES_SIZE), jnp.float32),
        compiler_params=pltpu.CompilerParams(
            disable_bounds_checks=True,
            disable_semaphore_checks=True,
            skip_device_barrier=True,
        ),
    )(ident, table, wt, b)


# speedup vs baseline: 1.0962x; 1.0962x over previous
import jax
import jax.numpy as jnp
from jax.experimental import pallas as pl
from jax.experimental.pallas import tpu as pltpu

_EMBED_DIM = 128
_RULES_SIZE = 1000
_ROWS_PER_BLOCK = 8


_CHUNKS = (256, 256, 256, 232)


def _fused_kernel(ident_ref, row_ref, wt_hbm, b_ref, out_ref, wt_vmem, *sems):
    copies = []
    off = 0
    for n, sem in zip(_CHUNKS, sems):
        c = pltpu.make_async_copy(
            wt_hbm.at[pl.ds(off, n), :], wt_vmem.at[pl.ds(off, n), :], sem)
        c.start()
        copies.append((off, n, c))
        off += n
    r = ident_ref[0] % _ROWS_PER_BLOCK
    row = row_ref[pl.ds(r, 1), :]
    for _, _, c in copies:
        c.wait()
    acc = jax.lax.dot_general(
        row, wt_vmem[...],
        dimension_numbers=(((1,), (1,)), ((), ())),
        preferred_element_type=jnp.float32,
    )
    out_ref[...] = jnp.maximum(acc + b_ref[...], 0.0)


def kernel(ident, table, W, b):
    ident = ident.astype(jnp.int32)
    wt = pltpu.with_memory_space_constraint(W.T, pltpu.MemorySpace.HBM)
    grid_spec = pltpu.PrefetchScalarGridSpec(
        num_scalar_prefetch=1,
        grid=(1,),
        in_specs=[
            pl.BlockSpec(
                (_ROWS_PER_BLOCK, _EMBED_DIM),
                lambda i, ident_ref: (ident_ref[0] // _ROWS_PER_BLOCK, 0),
            ),
            pl.BlockSpec(memory_space=pltpu.MemorySpace.HBM),
            pl.BlockSpec((_RULES_SIZE,), lambda i, ident_ref: (0,)),
        ],
        out_specs=pl.BlockSpec((1, _RULES_SIZE), lambda i, ident_ref: (0, 0)),
        scratch_shapes=[
            pltpu.VMEM((_RULES_SIZE, _EMBED_DIM), jnp.float32),
            pltpu.SemaphoreType.DMA,
            pltpu.SemaphoreType.DMA,
            pltpu.SemaphoreType.DMA,
            pltpu.SemaphoreType.DMA,
        ],
    )
    return pl.pallas_call(
        _fused_kernel,
        grid_spec=grid_spec,
        out_shape=jax.ShapeDtypeStruct((1, _RULES_SIZE), jnp.float32),
        compiler_params=pltpu.CompilerParams(
            disable_bounds_checks=True,
            disable_semaphore_checks=True,
            skip_device_barrier=True,
        ),
    )(ident, table, wt, b)
